# Initial kernel scaffold; baseline (speedup 1.0000x reference)
import jax
import jax.numpy as jnp
from jax.experimental import pallas as pl


def kernel(pixels, codebook, embeddings, codebook_vsa, position_roles):
    raise NotImplementedError("write your pallas kernel here")


# speedup vs baseline: 1.4428x; 1.4428x over previous
import functools

import jax
import jax.numpy as jnp
from jax import lax
from jax.experimental import pallas as pl
from jax.experimental.pallas import tpu as pltpu
from jax.experimental.pallas import tpu_sc as plsc

B = 128
K = 8192
PS = 8
IMG = 64
NPATCH = 64
PD = 192
ED = 2048
VD = 2048

_AGENT = [r * 8 + c for r in range(3, 6) for c in range(3, 6)]
_CENTRAL = [r * 8 + c for r in range(2, 6) for c in range(2, 6)]
_PERM = _AGENT + [p for p in _CENTRAL if p not in _AGENT] + [
    p for p in range(NPATCH) if p not in _CENTRAL
]
assert len(_PERM) == NPATCH and sorted(_PERM) == list(range(NPATCH))

_M_BLK = 256


def _tc_body(p_ref, cb_ref, out_ref):
    p = p_ref[...]
    s = jnp.sum(p * p, axis=1, keepdims=True)
    n = jnp.maximum(jnp.sqrt(s), 1e-8)
    a = (p / n).astype(jnp.bfloat16)
    sim = lax.dot_general(
        a, cb_ref[...], (((1,), (1,)), ((), ())),
        preferred_element_type=jnp.float32)
    out_ref[...] = jnp.argmax(sim, axis=1).astype(jnp.int32)


def _tc_indices(patches, cb16):
    return pl.pallas_call(
        _tc_body,
        grid=(B * NPATCH // _M_BLK,),
        in_specs=[
            pl.BlockSpec((_M_BLK, PD), lambda i: (i, 0)),
            pl.BlockSpec((K, PD), lambda i: (0, 0)),
        ],
        out_specs=pl.BlockSpec((_M_BLK,), lambda i: (i,)),
        out_shape=jax.ShapeDtypeStruct((B * NPATCH,), jnp.int32),
    )(patches, cb16)


_NC = 2
_NS = 16
_NW = _NC * _NS
_BPW = B // _NW
_NCHUNK = ED // 16


def _sc_body(idx_hbm, emb_hbm, vsa_hbm, roles_hbm,
             zr_hbm, zv_hbm, zl_hbm,
             gidx_v, rows_v, vsa_v, pos_v,
             zr_v, zv_v, zl_v, sem_e, sem_v):
    wid = lax.axis_index("s") * _NC + lax.axis_index("c")

    pltpu.sync_copy(roles_hbm, pos_v)

    for k in range(_BPW):
        b = wid * _BPW + k
        pltpu.sync_copy(idx_hbm.at[pl.ds(b * NPATCH, NPATCH)], gidx_v)

        vsa_cp = pltpu.async_copy(
            vsa_hbm.at[gidx_v.at[pl.ds(0, 16)]], vsa_v, sem_v)

        for g in range(4):
            pltpu.async_copy(
                emb_hbm.at[gidx_v.at[pl.ds(g * 16, 16)]], rows_v, sem_e
            ).wait()

            def fold(c, _, g=g):
                ds = pl.ds(c * 16, 16)
                s = rows_v[0, ds]
                for r in range(1, 9):
                    s = s + rows_v[r, ds]
                s9 = s
                for r in range(9, 16):
                    s = s + rows_v[r, ds]
                if g == 0:
                    zl_v[ds] = s9 * (1.0 / 9.0)
                    zr_v[ds] = s
                elif g < 3:
                    zr_v[ds] = zr_v[ds] + s
                else:
                    zr_v[ds] = (zr_v[ds] + s) * (1.0 / 64.0)
                return 0

            lax.fori_loop(0, _NCHUNK, fold, 0)

        vsa_cp.wait()

        def vsa_fold(c, _):
            ds = pl.ds(c * 16, 16)
            acc = jnp.where(vsa_v[0, ds] != pos_v[0, ds], 1.0, 0.0)
            for r in range(1, 16):
                acc = acc + jnp.where(vsa_v[r, ds] != pos_v[r, ds], 1.0, 0.0)
            zv_v[ds] = jnp.where(acc > 8.0, 1.0, 0.0)
            return 0

        lax.fori_loop(0, _NCHUNK, vsa_fold, 0)

        pltpu.sync_copy(zr_v, zr_hbm.at[b])
        pltpu.sync_copy(zv_v, zv_hbm.at[b])
        pltpu.sync_copy(zl_v, zl_hbm.at[b])


def _sc_gather(idx_perm, embeddings, codebook_vsa, roles_central):
    f32 = jnp.float32
    out_type = (
        jax.ShapeDtypeStruct((B, ED), f32),
        jax.ShapeDtypeStruct((B, VD), f32),
        jax.ShapeDtypeStruct((B, ED), f32),
    )
    mesh = plsc.VectorSubcoreMesh(
        core_axis_name="c", subcore_axis_name="s",
        num_cores=_NC, num_subcores=_NS)
    kfn = pl.kernel(
        _sc_body,
        out_type=out_type,
        mesh=mesh,
        scratch_types=[
            pltpu.VMEM((NPATCH,), jnp.int32),
            pltpu.VMEM((16, ED), f32),
            pltpu.VMEM((16, VD), f32),
            pltpu.VMEM((16, VD), f32),
            pltpu.VMEM((ED,), f32),
            pltpu.VMEM((VD,), f32),
            pltpu.VMEM((ED,), f32),
            pltpu.SemaphoreType.DMA,
            pltpu.SemaphoreType.DMA,
        ],
    )
    return kfn(idx_perm, embeddings, codebook_vsa, roles_central)


def kernel(pixels, codebook, embeddings, codebook_vsa, position_roles):
    x = pixels.reshape(B, 3, IMG // PS, PS, IMG // PS, PS)
    x = jnp.transpose(x, (0, 2, 4, 1, 3, 5))
    patches = x.reshape(B * NPATCH, PD)
    cb16 = codebook.astype(jnp.bfloat16)
    idx_flat = _tc_indices(patches, cb16)
    perm = jnp.asarray(_PERM, dtype=jnp.int32)
    idx_perm = idx_flat.reshape(B, NPATCH)[:, perm].reshape(B * NPATCH)
    roles_central = position_roles[jnp.asarray(_PERM[:16], dtype=jnp.int32)]
    z_real, z_vsa, z_local = _sc_gather(
        idx_perm, embeddings, codebook_vsa, roles_central)
    return z_real, z_vsa, idx_flat.reshape(B, NPATCH), z_local


# speedup vs baseline: 1.4560x; 1.0092x over previous
import functools

import jax
import jax.numpy as jnp
from jax import lax
from jax.experimental import pallas as pl
from jax.experimental.pallas import tpu as pltpu
from jax.experimental.pallas import tpu_sc as plsc

B = 128
K = 8192
PS = 8
IMG = 64
NPATCH = 64
PD = 192
ED = 2048
VD = 2048

_AGENT = [r * 8 + c for r in range(3, 6) for c in range(3, 6)]
_CENTRAL = [r * 8 + c for r in range(2, 6) for c in range(2, 6)]
_PERM = _AGENT + [p for p in _CENTRAL if p not in _AGENT] + [
    p for p in range(NPATCH) if p not in _CENTRAL
]
assert len(_PERM) == NPATCH and sorted(_PERM) == list(range(NPATCH))

_M_BLK = 256


def _tc_body(p_ref, cb_ref, out_ref):
    p = p_ref[...]
    s = jnp.sum(p * p, axis=1, keepdims=True)
    n = jnp.maximum(jnp.sqrt(s), 1e-8)
    a = (p / n).astype(jnp.bfloat16)
    sim = lax.dot_general(
        a, cb_ref[...], (((1,), (1,)), ((), ())),
        preferred_element_type=jnp.float32)
    out_ref[...] = jnp.argmax(sim, axis=1).astype(jnp.int32)


def _tc_indices(patches, cb16):
    return pl.pallas_call(
        _tc_body,
        grid=(B * NPATCH // _M_BLK,),
        in_specs=[
            pl.BlockSpec((_M_BLK, PD), lambda i: (i, 0)),
            pl.BlockSpec((K, PD), lambda i: (0, 0)),
        ],
        out_specs=pl.BlockSpec((_M_BLK,), lambda i: (i,)),
        out_shape=jax.ShapeDtypeStruct((B * NPATCH,), jnp.int32),
    )(patches, cb16)


_NC = 2
_NS = 16
_NW = _NC * _NS
_BPW = B // _NW
_NCHUNK = ED // 16


def _sc_body(idx_hbm, emb_hbm, vsa_hbm, roles_hbm, perm_hbm,
             zr_hbm, zv_hbm, zl_hbm,
             perm_v, pidx_v, gidx0_v, gidx1_v, buf_a, buf_b, pos_v,
             zr_v, zv_v, zl_v, sem_a, sem_b, sem_i):
    wid = lax.axis_index("s") * _NC + lax.axis_index("c")

    pltpu.sync_copy(perm_hbm, perm_v)
    pltpu.async_copy(
        roles_hbm.at[perm_v.at[pl.ds(0, 16)]], pos_v, sem_i).wait()

    gbufs = [gidx0_v, gidx1_v]

    def stage_gidx(k):
        base = (wid * _BPW + k) * NPATCH
        for g in range(4):
            ds = pl.ds(g * 16, 16)
            pidx_v[ds] = perm_v[ds] + base
        return pltpu.async_copy(idx_hbm.at[pidx_v], gbufs[k % 2], sem_i)

    def emb_fold(buf, g):
        def fold(c, _):
            ds = pl.ds(c * 16, 16)
            s = buf[0, ds]
            for r in range(1, 9):
                s = s + buf[r, ds]
            s9 = s
            for r in range(9, 16):
                s = s + buf[r, ds]
            if g == 0:
                zl_v[ds] = s9 * (1.0 / 9.0)
                zr_v[ds] = s
            elif g < 3:
                zr_v[ds] = zr_v[ds] + s
            else:
                zr_v[ds] = (zr_v[ds] + s) * (1.0 / 64.0)
            return 0

        lax.fori_loop(0, _NCHUNK, fold, 0)

    def vsa_fold(buf):
        def fold(c, _):
            ds = pl.ds(c * 16, 16)
            acc = jnp.where(buf[0, ds] != pos_v[0, ds], 1.0, 0.0)
            for r in range(1, 16):
                acc = acc + jnp.where(buf[r, ds] != pos_v[r, ds], 1.0, 0.0)
            zv_v[ds] = jnp.where(acc > 8.0, 1.0, 0.0)
            return 0

        lax.fori_loop(0, _NCHUNK, fold, 0)

    stage_gidx(0).wait()
    for k in range(_BPW):
        b = wid * _BPW + k
        gidx = gbufs[k % 2]
        cp_a = pltpu.async_copy(
            emb_hbm.at[gidx.at[pl.ds(0, 16)]], buf_a, sem_a)
        cp_b = pltpu.async_copy(
            emb_hbm.at[gidx.at[pl.ds(16, 16)]], buf_b, sem_b)
        cp_a.wait()
        emb_fold(buf_a, 0)
        cp_a = pltpu.async_copy(
            emb_hbm.at[gidx.at[pl.ds(32, 16)]], buf_a, sem_a)
        cp_b.wait()
        emb_fold(buf_b, 1)
        cp_b = pltpu.async_copy(
            emb_hbm.at[gidx.at[pl.ds(48, 16)]], buf_b, sem_b)
        cp_a.wait()
        emb_fold(buf_a, 2)
        cp_a = pltpu.async_copy(
            vsa_hbm.at[gidx.at[pl.ds(0, 16)]], buf_a, sem_a)
        cp_i = stage_gidx(k + 1) if k + 1 < _BPW else None
        cp_b.wait()
        emb_fold(buf_b, 3)
        cp_a.wait()
        vsa_fold(buf_a)
        if cp_i is not None:
            cp_i.wait()

        pltpu.sync_copy(zr_v, zr_hbm.at[b])
        pltpu.sync_copy(zv_v, zv_hbm.at[b])
        pltpu.sync_copy(zl_v, zl_hbm.at[b])


def _sc_gather(idx_flat, embeddings, codebook_vsa, position_roles, perm):
    f32 = jnp.float32
    i32 = jnp.int32
    out_type = (
        jax.ShapeDtypeStruct((B, ED), f32),
        jax.ShapeDtypeStruct((B, VD), f32),
        jax.ShapeDtypeStruct((B, ED), f32),
    )
    mesh = plsc.VectorSubcoreMesh(
        core_axis_name="c", subcore_axis_name="s",
        num_cores=_NC, num_subcores=_NS)
    kfn = pl.kernel(
        _sc_body,
        out_type=out_type,
        mesh=mesh,
        scratch_types=[
            pltpu.VMEM((NPATCH,), i32),
            pltpu.VMEM((NPATCH,), i32),
            pltpu.VMEM((NPATCH,), i32),
            pltpu.VMEM((NPATCH,), i32),
            pltpu.VMEM((16, ED), f32),
            pltpu.VMEM((16, ED), f32),
            pltpu.VMEM((16, VD), f32),
            pltpu.VMEM((ED,), f32),
            pltpu.VMEM((VD,), f32),
            pltpu.VMEM((ED,), f32),
            pltpu.SemaphoreType.DMA,
            pltpu.SemaphoreType.DMA,
            pltpu.SemaphoreType.DMA,
        ],
    )
    return kfn(idx_flat, embeddings, codebook_vsa, position_roles, perm)


def kernel(pixels, codebook, embeddings, codebook_vsa, position_roles):
    x = pixels.reshape(B, 3, IMG // PS, PS, IMG // PS, PS)
    x = jnp.transpose(x, (0, 2, 4, 1, 3, 5))
    patches = x.reshape(B * NPATCH, PD)
    cb16 = codebook.astype(jnp.bfloat16)
    idx_flat = _tc_indices(patches, cb16)
    perm = jnp.asarray(_PERM, dtype=jnp.int32)
    z_real, z_vsa, z_local = _sc_gather(
        idx_flat, embeddings, codebook_vsa, position_roles, perm)
    return z_real, z_vsa, idx_flat.reshape(B, NPATCH), z_local
